# baseline (speedup 1.0000x reference)
import functools

import jax
import jax.numpy as jnp
from jax import lax
from jax.experimental import pallas as pl
from jax.experimental.pallas import tpu as pltpu
from jax.experimental.pallas import tpu_sc as plsc

NC = 2
NS = 16
NW = NC * NS
L = 16

B = 16384
D = 32
BPW = B // NW
CH = 128
NCH = BPW // CH

NE = 1000000
NR = 1000

_mesh = plsc.VectorSubcoreMesh(core_axis_name="c", subcore_axis_name="s")


@functools.partial(
    pl.kernel,
    mesh=_mesh,
    out_type=jax.ShapeDtypeStruct((B * D,), jnp.float32),
    scratch_types=[
        pltpu.VMEM((16, CH), jnp.int32),
        pltpu.VMEM((CH, 128), jnp.float32),
        pltpu.VMEM((CH, 128), jnp.float32),
        pltpu.VMEM((CH, 128), jnp.float32),
        pltpu.VMEM((CH * D,), jnp.float32),
        pltpu.SemaphoreType.DMA,
    ],
    compiler_params=pltpu.CompilerParams(use_tc_tiling_on_sc=True,
                                         needs_layout_passes=False),
)
def _ptranse_sc(e1_hbm, e2_hbm, r_hbm, ent_hbm, rel_hbm, out_hbm,
                ib, g1, g2, gr, ob, sem):
    wid = lax.axis_index("s") * NC + lax.axis_index("c")
    base = wid * BPW

    iota = lax.iota(jnp.int32, L)

    for j in range(NCH):
        pltpu.sync_copy(e1_hbm.at[pl.ds(base + j * CH, CH)], ib.at[0])
        pltpu.sync_copy(e2_hbm.at[pl.ds(base + j * CH, CH)], ib.at[1])
        pltpu.sync_copy(r_hbm.at[pl.ds(base + j * CH, CH)], ib.at[2])
        for t in range(3):
            for l in range(CH // L):
                s = pl.ds(l * L, L)
                v = ib[t, s]
                ib[3 + t, s] = lax.shift_right_logical(v, 2)
                ib[6 + t, s] = lax.shift_left(lax.bitwise_and(v, 3), 5)

        c1 = pltpu.async_copy(ent_hbm.at[ib.at[3]], g1, sem)
        c2 = pltpu.async_copy(ent_hbm.at[ib.at[4]], g2, sem)
        c3 = pltpu.async_copy(rel_hbm.at[ib.at[5]], gr, sem)
        c1.wait()
        c2.wait()
        c3.wait()

        def group(g, _):
            rvec = g * L + iota
            obase = lax.shift_left(rvec, 5)
            s = pl.ds(g * L, L)
            s1 = ib[6, s]
            s2 = ib[7, s]
            sr = ib[8, s]
            for d in range(D):
                a = plsc.load_gather(g1, [rvec, s1 + d])
                c = plsc.load_gather(g2, [rvec, s2 + d])
                b = plsc.load_gather(gr, [rvec, sr + d])
                plsc.store_scatter(ob, [obase + d], jnp.abs(a + b - c))
            return 0

        lax.fori_loop(0, CH // L, group, 0)

        pltpu.sync_copy(ob, out_hbm.at[pl.ds((base + j * CH) * D, CH * D)])


def kernel(e1, e2, r, entity_emb, rel_emb):
    ent4 = entity_emb.reshape(NE // 4, 128)
    rel4 = rel_emb.reshape(NR // 4, 128)
    out = _ptranse_sc(e1.astype(jnp.int32), e2.astype(jnp.int32),
                      r.astype(jnp.int32), ent4, rel4)
    return out.reshape(B, D)


# speedup vs baseline: 1.0523x; 1.0523x over previous
import functools

import jax
import jax.numpy as jnp
from jax import lax
from jax.experimental import pallas as pl
from jax.experimental.pallas import tpu as pltpu
from jax.experimental.pallas import tpu_sc as plsc

NC = 2
NS = 16
NW = NC * NS
L = 16

B = 16384
D = 32
BPW = B // NW
CH = 128
NCH = BPW // CH

NE = 1000000
NR = 1000

_mesh = plsc.VectorSubcoreMesh(core_axis_name="c", subcore_axis_name="s")


@functools.partial(
    pl.kernel,
    mesh=_mesh,
    out_type=jax.ShapeDtypeStruct((D, B), jnp.float32),
    scratch_types=[
        pltpu.VMEM((BPW,), jnp.int32),
        pltpu.VMEM((BPW,), jnp.int32),
        pltpu.VMEM((BPW,), jnp.int32),
        pltpu.VMEM((BPW, D), jnp.float32),
        pltpu.VMEM((BPW, D), jnp.float32),
        pltpu.VMEM((NR, D), jnp.float32),
        pltpu.VMEM((D, BPW), jnp.float32),
        pltpu.SemaphoreType.DMA,
    ],
    compiler_params=pltpu.CompilerParams(use_tc_tiling_on_sc=False,
                                         needs_layout_passes=False),
)
def _ptranse_sc(e1_hbm, e2_hbm, r_hbm, ent_hbm, rel_hbm, out_hbm,
                i1, i2, ir, g1, g2, relv, ob, sem):
    w = lax.axis_index("s") * NC + lax.axis_index("c")
    base = w * BPW

    pltpu.sync_copy(e1_hbm.at[pl.ds(base, BPW)], i1)
    pltpu.sync_copy(e2_hbm.at[pl.ds(base, BPW)], i2)
    pltpu.sync_copy(r_hbm.at[pl.ds(base, BPW)], ir)
    pltpu.sync_copy(rel_hbm, relv)

    for j in range(NCH):
        pltpu.async_copy(ent_hbm.at[i1.at[pl.ds(j * CH, CH)]],
                         g1.at[pl.ds(j * CH, CH)], sem)
        pltpu.async_copy(ent_hbm.at[i2.at[pl.ds(j * CH, CH)]],
                         g2.at[pl.ds(j * CH, CH)], sem)
    for j in range(NCH):
        pltpu.make_async_copy(ent_hbm.at[i1.at[pl.ds(j * CH, CH)]],
                              g1.at[pl.ds(j * CH, CH)], sem).wait()
        pltpu.make_async_copy(ent_hbm.at[i2.at[pl.ds(j * CH, CH)]],
                              g2.at[pl.ds(j * CH, CH)], sem).wait()

    iota = lax.iota(jnp.int32, L)

    def group(l, _):
        s = pl.ds(l * L, L)
        bvec = l * L + iota
        rvec = ir[s]
        for d in range(D):
            dvec = jnp.full((L,), d, jnp.int32)
            a = plsc.load_gather(g1, [bvec, dvec])
            c = plsc.load_gather(g2, [bvec, dvec])
            rv = plsc.load_gather(relv, [rvec, dvec])
            ob[d, s] = jnp.abs(a + rv - c)
        return 0

    lax.fori_loop(0, BPW // L, group, 0)

    pltpu.sync_copy(ob, out_hbm.at[:, pl.ds(base, BPW)])


def kernel(e1, e2, r, entity_emb, rel_emb):
    out = _ptranse_sc(e1.astype(jnp.int32), e2.astype(jnp.int32),
                      r.astype(jnp.int32), entity_emb, rel_emb)
    return out.T


# speedup vs baseline: 1.1023x; 1.0475x over previous
import functools

import jax
import jax.numpy as jnp
from jax import lax
from jax.experimental import pallas as pl
from jax.experimental.pallas import tpu as pltpu
from jax.experimental.pallas import tpu_sc as plsc

NC = 2
NS = 16
NW = NC * NS
L = 16

B = 16384
D = 32
BPW = B // NW
CH = 128
NCH = BPW // CH

_mesh = plsc.VectorSubcoreMesh(core_axis_name="c", subcore_axis_name="s")


@functools.partial(
    pl.kernel,
    mesh=_mesh,
    out_type=jax.ShapeDtypeStruct((NW, NCH, CH, D), jnp.float32),
    scratch_types=[
        pltpu.VMEM((NCH, CH), jnp.int32),
        pltpu.VMEM((NCH, CH), jnp.int32),
        pltpu.VMEM((NCH, CH), jnp.int32),
        pltpu.VMEM((NCH, CH, D), jnp.float32),
        pltpu.VMEM((NCH, CH, D), jnp.float32),
        pltpu.VMEM((NCH, CH, D), jnp.float32),
        pltpu.SemaphoreType.DMA,
    ],
    compiler_params=pltpu.CompilerParams(use_tc_tiling_on_sc=False),
)
def _ptranse_sc(e1_hbm, e2_hbm, r_hbm, ent_hbm, rel_hbm, out_hbm,
                i1, i2, ir, r1, r2, rr, sem):
    wid = lax.axis_index("s") * NC + lax.axis_index("c")

    pltpu.sync_copy(e1_hbm.at[wid], i1)
    pltpu.sync_copy(e2_hbm.at[wid], i2)
    pltpu.sync_copy(r_hbm.at[wid], ir)

    copies = []
    for j in range(NCH):
        copies.append(pltpu.async_copy(ent_hbm.at[i1.at[j]], r1.at[j], sem))
        copies.append(pltpu.async_copy(ent_hbm.at[i2.at[j]], r2.at[j], sem))
        copies.append(pltpu.async_copy(rel_hbm.at[ir.at[j]], rr.at[j], sem))
    for c in copies:
        c.wait()

    def row_body(i, _):
        for j in range(NCH):
            for h in range(D // L):
                s = pl.ds(h * L, L)
                r1[j, i, s] = jnp.abs(r1[j, i, s] + rr[j, i, s] - r2[j, i, s])
        return 0

    lax.fori_loop(0, CH, row_body, 0)

    pltpu.sync_copy(r1, out_hbm.at[wid])


def kernel(e1, e2, r, entity_emb, rel_emb):
    e1w = e1.astype(jnp.int32).reshape(NW, NCH, CH)
    e2w = e2.astype(jnp.int32).reshape(NW, NCH, CH)
    rw = r.astype(jnp.int32).reshape(NW, NCH, CH)
    out = _ptranse_sc(e1w, e2w, rw, entity_emb, rel_emb)
    return out.reshape(B, D)
